# baseline (speedup 1.0000x reference)
import functools

import jax
import jax.numpy as jnp
from jax import lax
from jax.experimental import pallas as pl
from jax.experimental.pallas import tpu as pltpu
from jax.experimental.pallas import tpu_sc as plsc

_THR = 812
_BIL_B = 1024
_N = 10000
_NP = 10240
_H = 5120
_EC = 128


def _zero_block(zbuf, rows):
    zeros = jnp.zeros((16,), jnp.float32)

    def body(t, _):
        i = t // 8
        j = (t % 8) * 16
        zbuf[i, pl.ds(j, 16)] = zeros
        return 0

    lax.fori_loop(0, rows * 8, body, 0)


def _deg_kernel(col_pad):
    e_eff = col_pad.shape[0]
    per_tile = e_eff // 16
    nchunk = per_tile // _EC
    mesh = plsc.VectorSubcoreMesh(core_axis_name="c", subcore_axis_name="s")

    @functools.partial(
        pl.kernel,
        out_type=jax.ShapeDtypeStruct((_NP,), jnp.float32),
        mesh=mesh,
        scratch_types=[
            pltpu.VMEM((_EC,), jnp.int32),
            pltpu.VMEM((_EC,), jnp.int32),
            pltpu.VMEM((_EC,), jnp.float32),
            pltpu.VMEM((320,), jnp.float32),
            pltpu.VMEM_SHARED((_H + 8,), jnp.float32),
        ],
    )
    def k(col_hbm, deg_out, cbuf, dbuf, onesb, z1, deg_sp):
        core = lax.axis_index("c")
        tid = lax.axis_index("s")
        zeros = jnp.zeros((16,), jnp.float32)
        ones = jnp.ones((16,), jnp.float32)

        for j in range(320 // 16):
            z1[pl.ds(j * 16, 16)] = zeros
        for j in range(_EC // 16):
            onesb[pl.ds(j * 16, 16)] = ones
        pltpu.sync_copy(z1, deg_sp.at[pl.ds(tid * 320, 320)])

        @pl.when(tid == 0)
        def _():
            pltpu.sync_copy(z1.at[pl.ds(0, 8)], deg_sp.at[pl.ds(_H, 8)])

        plsc.subcore_barrier()
        base_lo = core * _H

        def chunk_a(ci, _):
            off = tid * per_tile + ci * _EC
            pltpu.sync_copy(col_hbm.at[pl.ds(off, _EC)], cbuf)
            for j in range(_EC // 16):
                c = cbuf[pl.ds(j * 16, 16)]
                lc = c - base_lo
                dcol = jnp.where((lc >= 0) & (lc < _H), lc, _H)
                dbuf[pl.ds(j * 16, 16)] = dcol
            pltpu.sync_copy(onesb, deg_sp.at[dbuf], add=True)
            return 0

        lax.fori_loop(0, nchunk, chunk_a, 0)
        plsc.subcore_barrier()

        pltpu.sync_copy(deg_sp.at[pl.ds(tid * 320, 320)], z1)
        pltpu.sync_copy(z1, deg_out.at[pl.ds(core * _H + tid * 320, 320)])

    return k(col_pad)


def _conv_accumulate(y_pad, row_pad, col_pad):
    e_eff = row_pad.shape[0]
    per_tile = e_eff // 16
    nchunk = per_tile // _EC
    mesh = plsc.VectorSubcoreMesh(core_axis_name="c", subcore_axis_name="s")

    @functools.partial(
        pl.kernel,
        out_type=[jax.ShapeDtypeStruct((_NP, 128), jnp.float32)] * 2,
        mesh=mesh,
        scratch_types=[
            pltpu.VMEM((_EC,), jnp.int32),
            pltpu.VMEM((_EC,), jnp.int32),
            pltpu.VMEM((_EC,), jnp.int32),
            pltpu.VMEM((_EC, 128), jnp.float32),
            pltpu.VMEM((128, 128), jnp.float32),
            pltpu.VMEM_SHARED((2 * _H + 16, 128), jnp.float32),
            pltpu.SemaphoreType.DMA,
        ],
    )
    def k(y_hbm, row_hbm, col_hbm, same_out, diff_out,
          rbuf, cbuf, dbuf, gbuf, zbuf, acc, sem):
        core = lax.axis_index("c")
        tid = lax.axis_index("s")

        _zero_block(zbuf, 128)
        for b in range(5):
            pltpu.sync_copy(zbuf, acc.at[pl.ds(tid * 640 + b * 128, 128)])

        @pl.when(tid == 0)
        def _():
            pltpu.sync_copy(zbuf.at[pl.ds(0, 16)], acc.at[pl.ds(2 * _H, 16)])

        plsc.subcore_barrier()

        base_lo = core * _H

        def chunk_body(ci, _):
            off = tid * per_tile + ci * _EC
            pltpu.sync_copy(row_hbm.at[pl.ds(off, _EC)], rbuf)
            pltpu.sync_copy(col_hbm.at[pl.ds(off, _EC)], cbuf)
            for j in range(_EC // 16):
                r = rbuf[pl.ds(j * 16, 16)]
                c = cbuf[pl.ds(j * 16, 16)]
                msame = ((r < _THR) & (c < _THR)) | ((r >= _THR) & (c >= _THR))
                lr = r - base_lo
                valid = (lr >= 0) & (lr < _H)
                dest = jnp.where(valid,
                                 jnp.where(msame, lr, lr + _H),
                                 2 * _H)
                dbuf[pl.ds(j * 16, 16)] = dest
            pltpu.async_copy(y_hbm.at[cbuf], gbuf, sem).wait()
            pltpu.sync_copy(gbuf, acc.at[dbuf], add=True)
            return 0

        lax.fori_loop(0, nchunk, chunk_body, 0)
        plsc.subcore_barrier()

        g0 = core * _H + tid * 320
        pltpu.sync_copy(acc.at[pl.ds(tid * 320, 320)], same_out.at[pl.ds(g0, 320)])
        pltpu.sync_copy(acc.at[pl.ds(_H + tid * 320, 320)], diff_out.at[pl.ds(g0, 320)])

    return k(y_pad, row_pad, col_pad)


def _transform_body(as_ref, ad_ref, dinv_ref, ws_ref, bs_ref, wd_ref, bd_ref,
                    xs_ref, xd_ref):
    scale = dinv_ref[...]
    a_s = as_ref[...] * scale
    a_d = ad_ref[...] * scale
    dn = (((1,), (1,)), ((), ()))
    xs_ref[...] = lax.dot_general(a_s, ws_ref[...], dn,
                                  preferred_element_type=jnp.float32) + bs_ref[...]
    xd_ref[...] = lax.dot_general(a_d, wd_ref[...], dn,
                                  preferred_element_type=jnp.float32) + bd_ref[...]


def _transform(acc_same, acc_diff, dinv, W_same, b_same, W_diff, b_diff, rb):
    n, d = acc_same.shape
    grid = (n // rb,)
    row_spec = pl.BlockSpec((rb, d), lambda i: (i, 0))
    one_spec = pl.BlockSpec((rb, 1), lambda i: (i, 0))
    w_spec = pl.BlockSpec((d, d), lambda i: (0, 0))
    b_spec = pl.BlockSpec((1, d), lambda i: (0, 0))
    return pl.pallas_call(
        _transform_body,
        grid=grid,
        in_specs=[row_spec, row_spec, one_spec, w_spec, b_spec, w_spec, b_spec],
        out_specs=[row_spec, row_spec],
        out_shape=[jax.ShapeDtypeStruct((n, d), jnp.float32)] * 2,
    )(acc_same, acc_diff, dinv[:, None], W_same, b_same[None, :],
      W_diff, b_diff[None, :])


def _bilinear_body(cnt_ref, f1_ref, f2_ref, w3_ref, bb_ref, eb_ref, *, b, d):
    i = pl.program_id(0)
    nb = (cnt_ref[0] + b - 1) // b

    @pl.when(i < nb)
    def _():
        f1b = f1_ref[...]
        f2b = f2_ref[...]
        acc = jnp.zeros((b, d), jnp.float32) + bb_ref[...]
        for t in range(d):
            a = f1b[:, t:t + 1] * f2b
            acc = acc + jnp.dot(a, w3_ref[t], preferred_element_type=jnp.float32)
        eb_ref[...] = acc


def _bilinear(cnt, f1p, f2p, W3, b_bil, b):
    e_pad, d = f1p.shape
    maxb = e_pad // b

    def edge_idx(i, c):
        nb = lax.div(c[0] + (b - 1), b)
        return (jnp.minimum(i, jnp.maximum(nb - 1, 0)), 0)

    grid_spec = pltpu.PrefetchScalarGridSpec(
        num_scalar_prefetch=1,
        grid=(maxb,),
        in_specs=[
            pl.BlockSpec((b, d), edge_idx),
            pl.BlockSpec((b, d), edge_idx),
            pl.BlockSpec((d, d, d), lambda i, c: (0, 0, 0)),
            pl.BlockSpec((1, d), lambda i, c: (0, 0)),
        ],
        out_specs=pl.BlockSpec((b, d), edge_idx),
    )
    return pl.pallas_call(
        functools.partial(_bilinear_body, b=b, d=d),
        grid_spec=grid_spec,
        out_shape=jax.ShapeDtypeStruct((e_pad, d), jnp.float32),
    )(cnt, f1p, f2p, W3, b_bil[None, :])


def _bil_scatter(eb, rows_m, cols_m, cnt16):
    e_pad = eb.shape[0]
    mesh = plsc.VectorSubcoreMesh(core_axis_name="c", subcore_axis_name="s")

    @functools.partial(
        pl.kernel,
        out_type=[jax.ShapeDtypeStruct((_NP, 128), jnp.float32),
                  jax.ShapeDtypeStruct((_NP,), jnp.float32)],
        mesh=mesh,
        scratch_types=[
            pltpu.VMEM((_EC,), jnp.int32),
            pltpu.VMEM((_EC,), jnp.int32),
            pltpu.VMEM((_EC,), jnp.int32),
            pltpu.VMEM((_EC,), jnp.int32),
            pltpu.VMEM((_EC,), jnp.float32),
            pltpu.VMEM((_EC, 128), jnp.float32),
            pltpu.VMEM((128, 128), jnp.float32),
            pltpu.VMEM((320,), jnp.float32),
            pltpu.VMEM((320,), jnp.float32),
            pltpu.VMEM((16,), jnp.int32),
            pltpu.VMEM_SHARED((_H + 8, 128), jnp.float32),
            pltpu.VMEM_SHARED((_H + 8,), jnp.float32),
            pltpu.SemaphoreType.DMA,
        ],
    )
    def k(eb_hbm, rows_hbm, cols_hbm, cnt_hbm, nb_out, cnts_out,
          rbuf, cbuf, dbr, dbc, onesb, gbuf, zbuf, z1, cbounce, cntbuf,
          nb_acc, cnt_acc, sem):
        core = lax.axis_index("c")
        tid = lax.axis_index("s")
        ones = jnp.ones((16,), jnp.float32)

        _zero_block(zbuf, 128)
        for j in range(_EC // 16):
            onesb[pl.ds(j * 16, 16)] = ones
        for j in range(320 // 16):
            z1[pl.ds(j * 16, 16)] = jnp.zeros((16,), jnp.float32)

        pltpu.sync_copy(zbuf, nb_acc.at[pl.ds(tid * 320, 128)])
        pltpu.sync_copy(zbuf, nb_acc.at[pl.ds(tid * 320 + 128, 128)])
        pltpu.sync_copy(zbuf.at[pl.ds(0, 64)], nb_acc.at[pl.ds(tid * 320 + 256, 64)])
        pltpu.sync_copy(z1, cnt_acc.at[pl.ds(tid * 320, 320)])

        @pl.when(tid == 0)
        def _():
            pltpu.sync_copy(zbuf.at[pl.ds(0, 8)], nb_acc.at[pl.ds(_H, 8)])
            pltpu.sync_copy(z1.at[pl.ds(0, 8)], cnt_acc.at[pl.ds(_H, 8)])

        pltpu.sync_copy(cnt_hbm, cntbuf)
        cv = cntbuf[pl.ds(0, 16)]
        cnt_s = cv[0]
        plsc.subcore_barrier()

        nch = (cnt_s + 16 * _EC - 1) // (16 * _EC)
        start = tid * nch * _EC
        trip = jnp.maximum(0, jnp.minimum(nch, (cnt_s - start + _EC - 1) // _EC))
        base_lo = core * _H

        def chunk_body(ci, _):
            off = start + ci * _EC
            pltpu.sync_copy(rows_hbm.at[pl.ds(off, _EC)], rbuf)
            pltpu.sync_copy(cols_hbm.at[pl.ds(off, _EC)], cbuf)
            pltpu.sync_copy(eb_hbm.at[pl.ds(off, _EC)], gbuf)
            cvl = cntbuf[pl.ds(0, 16)]
            for j in range(_EC // 16):
                gpos = off + j * 16 + lax.broadcasted_iota(jnp.int32, (16,), 0)
                ev = gpos < cvl
                r = rbuf[pl.ds(j * 16, 16)]
                c = cbuf[pl.ds(j * 16, 16)]
                lr = r - base_lo
                dr = jnp.where(ev & (lr >= 0) & (lr < _H), lr, _H)
                lc = c - base_lo
                dc = jnp.where(ev & (lc >= 0) & (lc < _H), lc, _H)
                dbr[pl.ds(j * 16, 16)] = dr
                dbc[pl.ds(j * 16, 16)] = dc
            pltpu.sync_copy(gbuf, nb_acc.at[dbr], add=True)
            pltpu.sync_copy(gbuf, nb_acc.at[dbc], add=True)
            pltpu.sync_copy(onesb, cnt_acc.at[dbr], add=True)
            pltpu.sync_copy(onesb, cnt_acc.at[dbc], add=True)
            return 0

        lax.fori_loop(0, trip, chunk_body, 0)
        plsc.subcore_barrier()

        g0 = core * _H + tid * 320
        pltpu.sync_copy(nb_acc.at[pl.ds(tid * 320, 320)], nb_out.at[pl.ds(g0, 320)])
        pltpu.sync_copy(cnt_acc.at[pl.ds(tid * 320, 320)], cbounce)
        pltpu.sync_copy(cbounce, cnts_out.at[pl.ds(g0, 320)])

    return k(eb, rows_m, cols_m, cnt16)


def _fuse_body(xs_ref, xd_ref, nb_ref, cnt_ref, gw_ref, out_ref):
    gate = 1.0 / (1.0 + jnp.exp(-gw_ref[...]))
    counts = jnp.maximum(cnt_ref[...], 1.0)
    bf = nb_ref[...] / counts
    leaky = jnp.where(bf >= 0, bf, 0.01 * bf)
    out_ref[...] = xs_ref[...] + gate * leaky + (1.0 - gate) * xd_ref[...]


def _fuse(x_same_t, x_diff_t, node_bil, counts, gate_weight, rb):
    n, d = x_same_t.shape
    row_spec = pl.BlockSpec((rb, d), lambda i: (i, 0))
    one_spec = pl.BlockSpec((rb, 1), lambda i: (i, 0))
    g_spec = pl.BlockSpec((1, d), lambda i: (0, 0))
    return pl.pallas_call(
        _fuse_body,
        grid=(n // rb,),
        in_specs=[row_spec, row_spec, row_spec, one_spec, g_spec],
        out_specs=row_spec,
        out_shape=jax.ShapeDtypeStruct((n, d), jnp.float32),
    )(x_same_t, x_diff_t, node_bil, counts[:, None], gate_weight[None, :])


def kernel(x, edge_index, W_same, b_same, W_diff, b_diff, W_bil, b_bil,
           gate_weight):
    n, d = x.shape
    e = edge_index.shape[1]
    rb = 1024
    row = edge_index[0]
    col = edge_index[1]

    e_eff = 161792
    row_pad = jnp.pad(row, (0, e_eff - e), constant_values=_N)
    col_pad = jnp.pad(col, (0, e_eff - e), constant_values=_N)
    deg = _deg_kernel(col_pad)
    dinv = deg ** -0.5

    x_pad = jnp.pad(x, ((0, _NP - n), (0, 0)))
    y_pad = dinv[:, None] * x_pad
    acc_same, acc_diff = _conv_accumulate(y_pad, row_pad, col_pad)

    x_same_t, x_diff_t = _transform(acc_same, acc_diff, dinv,
                                    W_same, b_same, W_diff, b_diff, rb=rb)

    m_md = (row < _THR) & (col >= _THR)
    e_pad = ((e + _BIL_B - 1) // _BIL_B) * _BIL_B
    mi = m_md.astype(jnp.int32)
    pos = jnp.cumsum(mi) - mi
    tgt = jnp.where(m_md, pos, e_pad)
    base = jnp.full((e_pad + 8,), _N, jnp.int32)
    rows_m = base.at[tgt].set(row, mode="drop")[:e_pad]
    cols_m = base.at[tgt].set(col, mode="drop")[:e_pad]
    cnt = pos[-1:] + mi[-1:]
    f1 = x_diff_t[rows_m]
    f2 = x_diff_t[cols_m]

    W3 = jnp.transpose(W_bil, (1, 2, 0))
    eb = _bilinear(cnt, f1, f2, W3, b_bil, _BIL_B)

    cnt16 = jnp.broadcast_to(cnt[0], (16,))
    node_bil, counts = _bil_scatter(eb, rows_m, cols_m, cnt16)

    out = _fuse(x_same_t, x_diff_t, node_bil, counts, gate_weight, rb=rb)
    return out[:n]


# speedup vs baseline: 1.2514x; 1.2514x over previous
import functools

import jax
import jax.numpy as jnp
from jax import lax
from jax.experimental import pallas as pl
from jax.experimental.pallas import tpu as pltpu
from jax.experimental.pallas import tpu_sc as plsc

_THR = 812
_BIL_B = 1024
_N = 10000
_NP = 10240
_H = 5120
_EC = 128


def _zero_block(zbuf, rows):
    zeros = jnp.zeros((16,), jnp.float32)

    def body(t, _):
        i = t // 8
        j = (t % 8) * 16
        zbuf[i, pl.ds(j, 16)] = zeros
        return 0

    lax.fori_loop(0, rows * 8, body, 0)


def _deg_kernel(col_pad):
    e_eff = col_pad.shape[0]
    per_tile = e_eff // 16
    nchunk = per_tile // _EC
    mesh = plsc.VectorSubcoreMesh(core_axis_name="c", subcore_axis_name="s")

    @functools.partial(
        pl.kernel,
        out_type=jax.ShapeDtypeStruct((_NP,), jnp.float32),
        mesh=mesh,
        scratch_types=[
            pltpu.VMEM((_EC,), jnp.int32),
            pltpu.VMEM((_EC,), jnp.int32),
            pltpu.VMEM((_EC,), jnp.float32),
            pltpu.VMEM((320,), jnp.float32),
            pltpu.VMEM_SHARED((_H + 8,), jnp.float32),
        ],
    )
    def k(col_hbm, deg_out, cbuf, dbuf, onesb, z1, deg_sp):
        core = lax.axis_index("c")
        tid = lax.axis_index("s")
        zeros = jnp.zeros((16,), jnp.float32)
        ones = jnp.ones((16,), jnp.float32)

        for j in range(320 // 16):
            z1[pl.ds(j * 16, 16)] = zeros
        for j in range(_EC // 16):
            onesb[pl.ds(j * 16, 16)] = ones
        pltpu.sync_copy(z1, deg_sp.at[pl.ds(tid * 320, 320)])

        @pl.when(tid == 0)
        def _():
            pltpu.sync_copy(z1.at[pl.ds(0, 8)], deg_sp.at[pl.ds(_H, 8)])

        plsc.subcore_barrier()
        base_lo = core * _H

        def chunk_a(ci, _):
            off = tid * per_tile + ci * _EC
            pltpu.sync_copy(col_hbm.at[pl.ds(off, _EC)], cbuf)
            for j in range(_EC // 16):
                c = cbuf[pl.ds(j * 16, 16)]
                lc = c - base_lo
                dcol = jnp.where((lc >= 0) & (lc < _H), lc, _H)
                dbuf[pl.ds(j * 16, 16)] = dcol
            pltpu.sync_copy(onesb, deg_sp.at[dbuf], add=True)
            return 0

        lax.fori_loop(0, nchunk, chunk_a, 0)
        plsc.subcore_barrier()

        pltpu.sync_copy(deg_sp.at[pl.ds(tid * 320, 320)], z1)
        pltpu.sync_copy(z1, deg_out.at[pl.ds(core * _H + tid * 320, 320)])

    return k(col_pad)


def _conv_accumulate(y_pad, row_pad, col_pad):
    e_eff = row_pad.shape[0]
    per_tile = e_eff // 16
    nchunk = per_tile // _EC
    mesh = plsc.VectorSubcoreMesh(core_axis_name="c", subcore_axis_name="s")

    @functools.partial(
        pl.kernel,
        out_type=[jax.ShapeDtypeStruct((_NP, 128), jnp.float32)] * 2,
        mesh=mesh,
        scratch_types=[
            pltpu.VMEM((_EC,), jnp.int32),
            pltpu.VMEM((_EC,), jnp.int32),
            pltpu.VMEM((_EC,), jnp.int32),
            pltpu.VMEM((_EC, 128), jnp.float32),
            pltpu.VMEM((128, 128), jnp.float32),
            pltpu.VMEM_SHARED((2 * _H + 16, 128), jnp.float32),
            pltpu.SemaphoreType.DMA,
        ],
    )
    def k(y_hbm, row_hbm, col_hbm, same_out, diff_out,
          rbuf, cbuf, dbuf, gbuf, zbuf, acc, sem):
        core = lax.axis_index("c")
        tid = lax.axis_index("s")

        _zero_block(zbuf, 128)
        for b in range(5):
            pltpu.sync_copy(zbuf, acc.at[pl.ds(tid * 640 + b * 128, 128)])

        @pl.when(tid == 0)
        def _():
            pltpu.sync_copy(zbuf.at[pl.ds(0, 16)], acc.at[pl.ds(2 * _H, 16)])

        plsc.subcore_barrier()

        base_lo = core * _H

        def chunk_body(ci, _):
            off = tid * per_tile + ci * _EC
            pltpu.sync_copy(row_hbm.at[pl.ds(off, _EC)], rbuf)
            pltpu.sync_copy(col_hbm.at[pl.ds(off, _EC)], cbuf)
            for j in range(_EC // 16):
                r = rbuf[pl.ds(j * 16, 16)]
                c = cbuf[pl.ds(j * 16, 16)]
                msame = ((r < _THR) & (c < _THR)) | ((r >= _THR) & (c >= _THR))
                lr = r - base_lo
                valid = (lr >= 0) & (lr < _H)
                dest = jnp.where(valid,
                                 jnp.where(msame, lr, lr + _H),
                                 2 * _H)
                dbuf[pl.ds(j * 16, 16)] = dest
            pltpu.async_copy(y_hbm.at[cbuf], gbuf, sem).wait()
            pltpu.sync_copy(gbuf, acc.at[dbuf], add=True)
            return 0

        lax.fori_loop(0, nchunk, chunk_body, 0)
        plsc.subcore_barrier()

        g0 = core * _H + tid * 320
        pltpu.sync_copy(acc.at[pl.ds(tid * 320, 320)], same_out.at[pl.ds(g0, 320)])
        pltpu.sync_copy(acc.at[pl.ds(_H + tid * 320, 320)], diff_out.at[pl.ds(g0, 320)])

    return k(y_pad, row_pad, col_pad)


def _transform_body(as_ref, ad_ref, dinv_ref, ws_ref, bs_ref, wd_ref, bd_ref,
                    xs_ref, xd_ref):
    scale = dinv_ref[...]
    a_s = as_ref[...] * scale
    a_d = ad_ref[...] * scale
    dn = (((1,), (1,)), ((), ()))
    xs_ref[...] = lax.dot_general(a_s, ws_ref[...], dn,
                                  preferred_element_type=jnp.float32) + bs_ref[...]
    xd_ref[...] = lax.dot_general(a_d, wd_ref[...], dn,
                                  preferred_element_type=jnp.float32) + bd_ref[...]


def _transform(acc_same, acc_diff, dinv, W_same, b_same, W_diff, b_diff, rb):
    n, d = acc_same.shape
    grid = (n // rb,)
    row_spec = pl.BlockSpec((rb, d), lambda i: (i, 0))
    one_spec = pl.BlockSpec((rb, 1), lambda i: (i, 0))
    w_spec = pl.BlockSpec((d, d), lambda i: (0, 0))
    b_spec = pl.BlockSpec((1, d), lambda i: (0, 0))
    return pl.pallas_call(
        _transform_body,
        grid=grid,
        in_specs=[row_spec, row_spec, one_spec, w_spec, b_spec, w_spec, b_spec],
        out_specs=[row_spec, row_spec],
        out_shape=[jax.ShapeDtypeStruct((n, d), jnp.float32)] * 2,
    )(acc_same, acc_diff, dinv[:, None], W_same, b_same[None, :],
      W_diff, b_diff[None, :])


def _bilinear_body(cnt_ref, f1_ref, f2_ref, w3_ref, bb_ref, eb_ref, *, b, d):
    i = pl.program_id(0)
    nb = (cnt_ref[0] + b - 1) // b

    @pl.when(i < nb)
    def _():
        f1b = f1_ref[...]
        f2b = f2_ref[...]
        acc = jnp.zeros((b, d), jnp.float32) + bb_ref[...]
        for t in range(d):
            a = f1b[:, t:t + 1] * f2b
            acc = acc + jnp.dot(a, w3_ref[t], preferred_element_type=jnp.float32)
        eb_ref[...] = acc


def _bilinear(cnt, f1p, f2p, W3, b_bil, b):
    e_pad, d = f1p.shape
    maxb = e_pad // b

    def edge_idx(i, c):
        nb = lax.div(c[0] + (b - 1), b)
        return (jnp.minimum(i, jnp.maximum(nb - 1, 0)), 0)

    grid_spec = pltpu.PrefetchScalarGridSpec(
        num_scalar_prefetch=1,
        grid=(maxb,),
        in_specs=[
            pl.BlockSpec((b, d), edge_idx),
            pl.BlockSpec((b, d), edge_idx),
            pl.BlockSpec((d, d, d), lambda i, c: (0, 0, 0)),
            pl.BlockSpec((1, d), lambda i, c: (0, 0)),
        ],
        out_specs=pl.BlockSpec((b, d), edge_idx),
    )
    return pl.pallas_call(
        functools.partial(_bilinear_body, b=b, d=d),
        grid_spec=grid_spec,
        out_shape=jax.ShapeDtypeStruct((e_pad, d), jnp.float32),
    )(cnt, f1p, f2p, W3, b_bil[None, :])


def _bil_scatter(eb, rows_m, cols_m, cnt16):
    e_pad = eb.shape[0]
    mesh = plsc.VectorSubcoreMesh(core_axis_name="c", subcore_axis_name="s")

    @functools.partial(
        pl.kernel,
        out_type=[jax.ShapeDtypeStruct((_NP, 128), jnp.float32),
                  jax.ShapeDtypeStruct((_NP,), jnp.float32)],
        mesh=mesh,
        scratch_types=[
            pltpu.VMEM((_EC,), jnp.int32),
            pltpu.VMEM((_EC,), jnp.int32),
            pltpu.VMEM((_EC,), jnp.int32),
            pltpu.VMEM((_EC,), jnp.int32),
            pltpu.VMEM((_EC,), jnp.float32),
            pltpu.VMEM((_EC, 128), jnp.float32),
            pltpu.VMEM((128, 128), jnp.float32),
            pltpu.VMEM((320,), jnp.float32),
            pltpu.VMEM((320,), jnp.float32),
            pltpu.VMEM((16,), jnp.int32),
            pltpu.VMEM_SHARED((_H + 8, 128), jnp.float32),
            pltpu.VMEM_SHARED((_H + 8,), jnp.float32),
            pltpu.SemaphoreType.DMA,
        ],
    )
    def k(eb_hbm, rows_hbm, cols_hbm, cnt_hbm, nb_out, cnts_out,
          rbuf, cbuf, dbr, dbc, onesb, gbuf, zbuf, z1, cbounce, cntbuf,
          nb_acc, cnt_acc, sem):
        core = lax.axis_index("c")
        tid = lax.axis_index("s")
        ones = jnp.ones((16,), jnp.float32)

        _zero_block(zbuf, 128)
        for j in range(_EC // 16):
            onesb[pl.ds(j * 16, 16)] = ones
        for j in range(320 // 16):
            z1[pl.ds(j * 16, 16)] = jnp.zeros((16,), jnp.float32)

        pltpu.sync_copy(zbuf, nb_acc.at[pl.ds(tid * 320, 128)])
        pltpu.sync_copy(zbuf, nb_acc.at[pl.ds(tid * 320 + 128, 128)])
        pltpu.sync_copy(zbuf.at[pl.ds(0, 64)], nb_acc.at[pl.ds(tid * 320 + 256, 64)])
        pltpu.sync_copy(z1, cnt_acc.at[pl.ds(tid * 320, 320)])

        @pl.when(tid == 0)
        def _():
            pltpu.sync_copy(zbuf.at[pl.ds(0, 8)], nb_acc.at[pl.ds(_H, 8)])
            pltpu.sync_copy(z1.at[pl.ds(0, 8)], cnt_acc.at[pl.ds(_H, 8)])

        pltpu.sync_copy(cnt_hbm, cntbuf)
        cv = cntbuf[pl.ds(0, 16)]
        cnt_s = cv[0]
        plsc.subcore_barrier()

        nch = (cnt_s + 16 * _EC - 1) // (16 * _EC)
        start = tid * nch * _EC
        trip = jnp.maximum(0, jnp.minimum(nch, (cnt_s - start + _EC - 1) // _EC))
        base_lo = core * _H

        def chunk_body(ci, _):
            off = start + ci * _EC
            pltpu.sync_copy(rows_hbm.at[pl.ds(off, _EC)], rbuf)
            pltpu.sync_copy(cols_hbm.at[pl.ds(off, _EC)], cbuf)
            pltpu.sync_copy(eb_hbm.at[pl.ds(off, _EC)], gbuf)
            cvl = cntbuf[pl.ds(0, 16)]
            for j in range(_EC // 16):
                gpos = off + j * 16 + lax.broadcasted_iota(jnp.int32, (16,), 0)
                ev = gpos < cvl
                r = rbuf[pl.ds(j * 16, 16)]
                c = cbuf[pl.ds(j * 16, 16)]
                lr = r - base_lo
                dr = jnp.where(ev & (lr >= 0) & (lr < _H), lr, _H)
                lc = c - base_lo
                dc = jnp.where(ev & (lc >= 0) & (lc < _H), lc, _H)
                dbr[pl.ds(j * 16, 16)] = dr
                dbc[pl.ds(j * 16, 16)] = dc
            pltpu.sync_copy(gbuf, nb_acc.at[dbr], add=True)
            pltpu.sync_copy(gbuf, nb_acc.at[dbc], add=True)
            pltpu.sync_copy(onesb, cnt_acc.at[dbr], add=True)
            pltpu.sync_copy(onesb, cnt_acc.at[dbc], add=True)
            return 0

        lax.fori_loop(0, trip, chunk_body, 0)
        plsc.subcore_barrier()

        g0 = core * _H + tid * 320
        pltpu.sync_copy(nb_acc.at[pl.ds(tid * 320, 320)], nb_out.at[pl.ds(g0, 320)])
        pltpu.sync_copy(cnt_acc.at[pl.ds(tid * 320, 320)], cbounce)
        pltpu.sync_copy(cbounce, cnts_out.at[pl.ds(g0, 320)])

    return k(eb, rows_m, cols_m, cnt16)


def _fuse_body(xs_ref, xd_ref, nb_ref, cnt_ref, gw_ref, out_ref):
    gate = 1.0 / (1.0 + jnp.exp(-gw_ref[...]))
    counts = jnp.maximum(cnt_ref[...], 1.0)
    bf = nb_ref[...] / counts
    leaky = jnp.where(bf >= 0, bf, 0.01 * bf)
    out_ref[...] = xs_ref[...] + gate * leaky + (1.0 - gate) * xd_ref[...]


def _fuse(x_same_t, x_diff_t, node_bil, counts, gate_weight, rb):
    n, d = x_same_t.shape
    row_spec = pl.BlockSpec((rb, d), lambda i: (i, 0))
    one_spec = pl.BlockSpec((rb, 1), lambda i: (i, 0))
    g_spec = pl.BlockSpec((1, d), lambda i: (0, 0))
    return pl.pallas_call(
        _fuse_body,
        grid=(n // rb,),
        in_specs=[row_spec, row_spec, row_spec, one_spec, g_spec],
        out_specs=row_spec,
        out_shape=jax.ShapeDtypeStruct((n, d), jnp.float32),
    )(x_same_t, x_diff_t, node_bil, counts[:, None], gate_weight[None, :])


def kernel(x, edge_index, W_same, b_same, W_diff, b_diff, W_bil, b_bil,
           gate_weight):
    n, d = x.shape
    e = edge_index.shape[1]
    rb = 1024
    row = edge_index[0]
    col = edge_index[1]

    e_eff = 161792
    row_pad = jnp.pad(row, (0, e_eff - e), constant_values=_N)
    col_pad = jnp.pad(col, (0, e_eff - e), constant_values=_N)
    deg = _deg_kernel(col_pad)
    dinv = deg ** -0.5

    x_pad = jnp.pad(x, ((0, _NP - n), (0, 0)))
    y_pad = dinv[:, None] * x_pad
    acc_same, acc_diff = _conv_accumulate(y_pad, row_pad, col_pad)

    x_same_t, x_diff_t = _transform(acc_same, acc_diff, dinv,
                                    W_same, b_same, W_diff, b_diff, rb=rb)

    m_md = (row < _THR) & (col >= _THR)
    order = jnp.argsort(jnp.logical_not(m_md), stable=True)
    e_pad = ((e + _BIL_B - 1) // _BIL_B) * _BIL_B
    pad = e_pad - e
    rows_m = jnp.pad(row[order], (0, pad))
    cols_m = jnp.pad(col[order], (0, pad))
    cnt = jnp.sum(m_md).astype(jnp.int32)[None]
    f1 = x_diff_t[rows_m]
    f2 = x_diff_t[cols_m]

    W3 = jnp.transpose(W_bil, (1, 2, 0)).astype(jnp.bfloat16)
    eb = _bilinear(cnt, f1.astype(jnp.bfloat16), f2.astype(jnp.bfloat16),
                   W3, b_bil, _BIL_B)

    cnt16 = jnp.broadcast_to(cnt[0], (16,))
    node_bil, counts = _bil_scatter(eb, rows_m, cols_m, cnt16)

    out = _fuse(x_same_t, x_diff_t, node_bil, counts, gate_weight, rb=rb)
    return out[:n]


# speedup vs baseline: 1.3506x; 1.0792x over previous
import functools

import jax
import jax.numpy as jnp
from jax import lax
from jax.experimental import pallas as pl
from jax.experimental.pallas import tpu as pltpu
from jax.experimental.pallas import tpu_sc as plsc

_THR = 812
_BIL_B = 1024
_N = 10000
_NP = 10240
_H = 5120
_EC = 128


def _zero_block(zbuf, rows):
    zeros = jnp.zeros((16,), jnp.float32)

    def body(t, _):
        i = t // 8
        j = (t % 8) * 16
        zbuf[i, pl.ds(j, 16)] = zeros
        return 0

    lax.fori_loop(0, rows * 8, body, 0)


def _deg_kernel(col_pad):
    e_eff = col_pad.shape[0]
    per_tile = e_eff // 16
    nchunk = per_tile // _EC
    mesh = plsc.VectorSubcoreMesh(core_axis_name="c", subcore_axis_name="s")

    @functools.partial(
        pl.kernel,
        out_type=jax.ShapeDtypeStruct((_NP,), jnp.float32),
        mesh=mesh,
        scratch_types=[
            pltpu.VMEM((_EC,), jnp.int32),
            pltpu.VMEM((_EC,), jnp.int32),
            pltpu.VMEM((_EC,), jnp.float32),
            pltpu.VMEM((320,), jnp.float32),
            pltpu.VMEM_SHARED((_H + 8,), jnp.float32),
        ],
    )
    def k(col_hbm, deg_out, cbuf, dbuf, onesb, z1, deg_sp):
        core = lax.axis_index("c")
        tid = lax.axis_index("s")
        zeros = jnp.zeros((16,), jnp.float32)
        ones = jnp.ones((16,), jnp.float32)

        for j in range(320 // 16):
            z1[pl.ds(j * 16, 16)] = zeros
        for j in range(_EC // 16):
            onesb[pl.ds(j * 16, 16)] = ones
        pltpu.sync_copy(z1, deg_sp.at[pl.ds(tid * 320, 320)])

        @pl.when(tid == 0)
        def _():
            pltpu.sync_copy(z1.at[pl.ds(0, 8)], deg_sp.at[pl.ds(_H, 8)])

        plsc.subcore_barrier()
        base_lo = core * _H

        def chunk_a(ci, _):
            off = tid * per_tile + ci * _EC
            pltpu.sync_copy(col_hbm.at[pl.ds(off, _EC)], cbuf)
            for j in range(_EC // 16):
                c = cbuf[pl.ds(j * 16, 16)]
                lc = c - base_lo
                dcol = jnp.where((lc >= 0) & (lc < _H), lc, _H)
                dbuf[pl.ds(j * 16, 16)] = dcol
            pltpu.sync_copy(onesb, deg_sp.at[dbuf], add=True)
            return 0

        lax.fori_loop(0, nchunk, chunk_a, 0)
        plsc.subcore_barrier()

        pltpu.sync_copy(deg_sp.at[pl.ds(tid * 320, 320)], z1)
        pltpu.sync_copy(z1, deg_out.at[pl.ds(core * _H + tid * 320, 320)])

    return k(col_pad)


def _conv_accumulate(y_pad, row_pad, col_pad):
    e_eff = row_pad.shape[0]
    per_tile = e_eff // 16
    nchunk = per_tile // _EC
    mesh = plsc.VectorSubcoreMesh(core_axis_name="c", subcore_axis_name="s")

    @functools.partial(
        pl.kernel,
        out_type=[jax.ShapeDtypeStruct((_NP, 128), jnp.float32)] * 2,
        mesh=mesh,
        scratch_types=[
            pltpu.VMEM((_EC,), jnp.int32),
            pltpu.VMEM((_EC,), jnp.int32),
            pltpu.VMEM((_EC,), jnp.int32),
            pltpu.VMEM((_EC, 128), jnp.float32),
            pltpu.VMEM((128, 128), jnp.float32),
            pltpu.VMEM_SHARED((2 * _H + 16, 128), jnp.float32),
            pltpu.SemaphoreType.DMA,
        ],
    )
    def k(y_hbm, row_hbm, col_hbm, same_out, diff_out,
          rbuf, cbuf, dbuf, gbuf, zbuf, acc, sem):
        core = lax.axis_index("c")
        tid = lax.axis_index("s")

        _zero_block(zbuf, 128)
        for b in range(5):
            pltpu.sync_copy(zbuf, acc.at[pl.ds(tid * 640 + b * 128, 128)])

        @pl.when(tid == 0)
        def _():
            pltpu.sync_copy(zbuf.at[pl.ds(0, 16)], acc.at[pl.ds(2 * _H, 16)])

        plsc.subcore_barrier()

        base_lo = core * _H

        def chunk_body(ci, _):
            off = tid * per_tile + ci * _EC
            pltpu.sync_copy(row_hbm.at[pl.ds(off, _EC)], rbuf)
            pltpu.sync_copy(col_hbm.at[pl.ds(off, _EC)], cbuf)
            for j in range(_EC // 16):
                r = rbuf[pl.ds(j * 16, 16)]
                c = cbuf[pl.ds(j * 16, 16)]
                msame = ((r < _THR) & (c < _THR)) | ((r >= _THR) & (c >= _THR))
                lr = r - base_lo
                valid = (lr >= 0) & (lr < _H)
                dest = jnp.where(valid,
                                 jnp.where(msame, lr, lr + _H),
                                 2 * _H)
                dbuf[pl.ds(j * 16, 16)] = dest
            pltpu.async_copy(y_hbm.at[cbuf], gbuf, sem).wait()
            pltpu.sync_copy(gbuf, acc.at[dbuf], add=True)
            return 0

        lax.fori_loop(0, nchunk, chunk_body, 0)
        plsc.subcore_barrier()

        g0 = core * _H + tid * 320
        pltpu.sync_copy(acc.at[pl.ds(tid * 320, 320)], same_out.at[pl.ds(g0, 320)])
        pltpu.sync_copy(acc.at[pl.ds(_H + tid * 320, 320)], diff_out.at[pl.ds(g0, 320)])

    return k(y_pad, row_pad, col_pad)


def _transform_body(as_ref, ad_ref, dinv_ref, ws_ref, bs_ref, wd_ref, bd_ref,
                    xs_ref, xd_ref):
    scale = dinv_ref[...]
    a_s = as_ref[...] * scale
    a_d = ad_ref[...] * scale
    dn = (((1,), (1,)), ((), ()))
    xs_ref[...] = lax.dot_general(a_s, ws_ref[...], dn,
                                  preferred_element_type=jnp.float32) + bs_ref[...]
    xd_ref[...] = lax.dot_general(a_d, wd_ref[...], dn,
                                  preferred_element_type=jnp.float32) + bd_ref[...]


def _transform(acc_same, acc_diff, dinv, W_same, b_same, W_diff, b_diff, rb):
    n, d = acc_same.shape
    grid = (n // rb,)
    row_spec = pl.BlockSpec((rb, d), lambda i: (i, 0))
    one_spec = pl.BlockSpec((rb, 1), lambda i: (i, 0))
    w_spec = pl.BlockSpec((d, d), lambda i: (0, 0))
    b_spec = pl.BlockSpec((1, d), lambda i: (0, 0))
    return pl.pallas_call(
        _transform_body,
        grid=grid,
        in_specs=[row_spec, row_spec, one_spec, w_spec, b_spec, w_spec, b_spec],
        out_specs=[row_spec, row_spec],
        out_shape=[jax.ShapeDtypeStruct((n, d), jnp.float32)] * 2,
    )(acc_same, acc_diff, dinv[:, None], W_same, b_same[None, :],
      W_diff, b_diff[None, :])


def _bilinear_body(cnt_ref, f1_ref, f2_ref, w3_ref, bb_ref, eb_ref, *, b, d):
    i = pl.program_id(0)
    nb = (cnt_ref[0] + b - 1) // b

    @pl.when(i < nb)
    def _():
        f1b = f1_ref[...]
        f2b = f2_ref[...]
        acc = jnp.zeros((b, d), jnp.float32) + bb_ref[...]
        for t in range(d):
            a = f1b[:, t:t + 1] * f2b
            acc = acc + jnp.dot(a, w3_ref[t], preferred_element_type=jnp.float32)
        eb_ref[...] = acc


def _bilinear(cnt, f1p, f2p, W3, b_bil, b):
    e_pad, d = f1p.shape
    maxb = e_pad // b

    def edge_idx(i, c):
        nb = lax.div(c[0] + (b - 1), b)
        return (jnp.minimum(i, jnp.maximum(nb - 1, 0)), 0)

    grid_spec = pltpu.PrefetchScalarGridSpec(
        num_scalar_prefetch=1,
        grid=(maxb,),
        in_specs=[
            pl.BlockSpec((b, d), edge_idx),
            pl.BlockSpec((b, d), edge_idx),
            pl.BlockSpec((d, d, d), lambda i, c: (0, 0, 0)),
            pl.BlockSpec((1, d), lambda i, c: (0, 0)),
        ],
        out_specs=pl.BlockSpec((b, d), edge_idx),
    )
    return pl.pallas_call(
        functools.partial(_bilinear_body, b=b, d=d),
        grid_spec=grid_spec,
        out_shape=jax.ShapeDtypeStruct((e_pad, d), jnp.float32),
    )(cnt, f1p, f2p, W3, b_bil[None, :])


def _bil_scatter(eb, rows_m, cols_m, cnt16):
    e_pad = eb.shape[0]
    mesh = plsc.VectorSubcoreMesh(core_axis_name="c", subcore_axis_name="s")

    @functools.partial(
        pl.kernel,
        out_type=[jax.ShapeDtypeStruct((_NP, 128), jnp.float32),
                  jax.ShapeDtypeStruct((_NP,), jnp.float32)],
        mesh=mesh,
        scratch_types=[
            pltpu.VMEM((_EC,), jnp.int32),
            pltpu.VMEM((_EC,), jnp.int32),
            pltpu.VMEM((_EC,), jnp.int32),
            pltpu.VMEM((_EC,), jnp.int32),
            pltpu.VMEM((_EC,), jnp.float32),
            pltpu.VMEM((_EC, 128), jnp.float32),
            pltpu.VMEM((128, 128), jnp.float32),
            pltpu.VMEM((320,), jnp.float32),
            pltpu.VMEM((320,), jnp.float32),
            pltpu.VMEM((16,), jnp.int32),
            pltpu.VMEM_SHARED((_H + 8, 128), jnp.float32),
            pltpu.VMEM_SHARED((_H + 8,), jnp.float32),
            pltpu.SemaphoreType.DMA,
        ],
    )
    def k(eb_hbm, rows_hbm, cols_hbm, cnt_hbm, nb_out, cnts_out,
          rbuf, cbuf, dbr, dbc, onesb, gbuf, zbuf, z1, cbounce, cntbuf,
          nb_acc, cnt_acc, sem):
        core = lax.axis_index("c")
        tid = lax.axis_index("s")
        ones = jnp.ones((16,), jnp.float32)

        _zero_block(zbuf, 128)
        for j in range(_EC // 16):
            onesb[pl.ds(j * 16, 16)] = ones
        for j in range(320 // 16):
            z1[pl.ds(j * 16, 16)] = jnp.zeros((16,), jnp.float32)

        pltpu.sync_copy(zbuf, nb_acc.at[pl.ds(tid * 320, 128)])
        pltpu.sync_copy(zbuf, nb_acc.at[pl.ds(tid * 320 + 128, 128)])
        pltpu.sync_copy(zbuf.at[pl.ds(0, 64)], nb_acc.at[pl.ds(tid * 320 + 256, 64)])
        pltpu.sync_copy(z1, cnt_acc.at[pl.ds(tid * 320, 320)])

        @pl.when(tid == 0)
        def _():
            pltpu.sync_copy(zbuf.at[pl.ds(0, 8)], nb_acc.at[pl.ds(_H, 8)])
            pltpu.sync_copy(z1.at[pl.ds(0, 8)], cnt_acc.at[pl.ds(_H, 8)])

        pltpu.sync_copy(cnt_hbm, cntbuf)
        cv = cntbuf[pl.ds(0, 16)]
        cnt_s = cv[0]
        plsc.subcore_barrier()

        nch = (cnt_s + 16 * _EC - 1) // (16 * _EC)
        start = tid * nch * _EC
        trip = jnp.maximum(0, jnp.minimum(nch, (cnt_s - start + _EC - 1) // _EC))
        base_lo = core * _H

        def chunk_body(ci, _):
            off = start + ci * _EC
            pltpu.sync_copy(rows_hbm.at[pl.ds(off, _EC)], rbuf)
            pltpu.sync_copy(cols_hbm.at[pl.ds(off, _EC)], cbuf)
            pltpu.sync_copy(eb_hbm.at[pl.ds(off, _EC)], gbuf)
            cvl = cntbuf[pl.ds(0, 16)]
            for j in range(_EC // 16):
                gpos = off + j * 16 + lax.broadcasted_iota(jnp.int32, (16,), 0)
                ev = gpos < cvl
                r = rbuf[pl.ds(j * 16, 16)]
                c = cbuf[pl.ds(j * 16, 16)]
                lr = r - base_lo
                dr = jnp.where(ev & (lr >= 0) & (lr < _H), lr, _H)
                lc = c - base_lo
                dc = jnp.where(ev & (lc >= 0) & (lc < _H), lc, _H)
                dbr[pl.ds(j * 16, 16)] = dr
                dbc[pl.ds(j * 16, 16)] = dc
            pltpu.sync_copy(gbuf, nb_acc.at[dbr], add=True)
            pltpu.sync_copy(gbuf, nb_acc.at[dbc], add=True)
            pltpu.sync_copy(onesb, cnt_acc.at[dbr], add=True)
            pltpu.sync_copy(onesb, cnt_acc.at[dbc], add=True)
            return 0

        lax.fori_loop(0, trip, chunk_body, 0)
        plsc.subcore_barrier()

        g0 = core * _H + tid * 320
        pltpu.sync_copy(nb_acc.at[pl.ds(tid * 320, 320)], nb_out.at[pl.ds(g0, 320)])
        pltpu.sync_copy(cnt_acc.at[pl.ds(tid * 320, 320)], cbounce)
        pltpu.sync_copy(cbounce, cnts_out.at[pl.ds(g0, 320)])

    return k(eb, rows_m, cols_m, cnt16)


def _fuse_body(xs_ref, xd_ref, nb_ref, cnt_ref, gw_ref, out_ref):
    gate = 1.0 / (1.0 + jnp.exp(-gw_ref[...]))
    counts = jnp.maximum(cnt_ref[...], 1.0)
    bf = nb_ref[...] / counts
    leaky = jnp.where(bf >= 0, bf, 0.01 * bf)
    out_ref[...] = xs_ref[...] + gate * leaky + (1.0 - gate) * xd_ref[...]


def _fuse(x_same_t, x_diff_t, node_bil, counts, gate_weight, rb):
    n, d = x_same_t.shape
    row_spec = pl.BlockSpec((rb, d), lambda i: (i, 0))
    one_spec = pl.BlockSpec((rb, 1), lambda i: (i, 0))
    g_spec = pl.BlockSpec((1, d), lambda i: (0, 0))
    return pl.pallas_call(
        _fuse_body,
        grid=(n // rb,),
        in_specs=[row_spec, row_spec, row_spec, one_spec, g_spec],
        out_specs=row_spec,
        out_shape=jax.ShapeDtypeStruct((n, d), jnp.float32),
    )(x_same_t, x_diff_t, node_bil, counts[:, None], gate_weight[None, :])


def kernel(x, edge_index, W_same, b_same, W_diff, b_diff, W_bil, b_bil,
           gate_weight):
    n, d = x.shape
    e = edge_index.shape[1]
    rb = 1024
    row = edge_index[0]
    col = edge_index[1]

    e_eff = 161792
    row_pad = jnp.pad(row, (0, e_eff - e), constant_values=_N)
    col_pad = jnp.pad(col, (0, e_eff - e), constant_values=_N)
    deg = _deg_kernel(col_pad)
    dinv = deg ** -0.5

    x_pad = jnp.pad(x, ((0, _NP - n), (0, 0)))
    y_pad = dinv[:, None] * x_pad
    acc_same, acc_diff = _conv_accumulate(y_pad, row_pad, col_pad)

    x_same_t, x_diff_t = _transform(acc_same, acc_diff, dinv,
                                    W_same, b_same, W_diff, b_diff, rb=rb)

    m_md = (row < _THR) & (col >= _THR)
    order = jnp.argsort(jnp.logical_not(m_md), stable=True)
    e_pad = ((e + _BIL_B - 1) // _BIL_B) * _BIL_B
    pad = e_pad - e
    rows_m = jnp.pad(row[order], (0, pad))
    cols_m = jnp.pad(col[order], (0, pad))
    cnt = jnp.sum(m_md).astype(jnp.int32)[None]
    f1 = x_diff_t[rows_m]
    f2 = x_diff_t[cols_m]

    W3 = jnp.transpose(W_bil, (1, 2, 0))
    eb = _bilinear(cnt, f1, f2, W3, b_bil, _BIL_B)

    cnt16 = jnp.broadcast_to(cnt[0], (16,))
    node_bil, counts = _bil_scatter(eb, rows_m, cols_m, cnt16)

    out = _fuse(x_same_t, x_diff_t, node_bil, counts, gate_weight, rb=rb)
    return out[:n]


# speedup vs baseline: 2.8073x; 2.0786x over previous
import functools

import jax
import jax.numpy as jnp
from jax import lax
from jax.experimental import pallas as pl
from jax.experimental.pallas import tpu as pltpu
from jax.experimental.pallas import tpu_sc as plsc

_THR = 812
_BIL_B = 1024
_N = 10000
_NP = 10240
_H = 5120
_EC = 128


def _zero_block(zbuf, rows):
    zeros = jnp.zeros((16,), jnp.float32)

    def body(t, _):
        i = t // 8
        j = (t % 8) * 16
        zbuf[i, pl.ds(j, 16)] = zeros
        return 0

    lax.fori_loop(0, rows * 8, body, 0)


def _deg_kernel(col_pad):
    e_eff = col_pad.shape[0]
    per_tile = e_eff // 16
    nchunk = per_tile // _EC
    mesh = plsc.VectorSubcoreMesh(core_axis_name="c", subcore_axis_name="s")

    @functools.partial(
        pl.kernel,
        out_type=jax.ShapeDtypeStruct((_NP,), jnp.float32),
        mesh=mesh,
        scratch_types=[
            pltpu.VMEM((_EC,), jnp.int32),
            pltpu.VMEM((_EC,), jnp.int32),
            pltpu.VMEM((_EC,), jnp.float32),
            pltpu.VMEM((320,), jnp.float32),
            pltpu.VMEM_SHARED((_H + 8,), jnp.float32),
        ],
    )
    def k(col_hbm, deg_out, cbuf, dbuf, onesb, z1, deg_sp):
        core = lax.axis_index("c")
        tid = lax.axis_index("s")
        zeros = jnp.zeros((16,), jnp.float32)
        ones = jnp.ones((16,), jnp.float32)

        for j in range(320 // 16):
            z1[pl.ds(j * 16, 16)] = zeros
        for j in range(_EC // 16):
            onesb[pl.ds(j * 16, 16)] = ones
        pltpu.sync_copy(z1, deg_sp.at[pl.ds(tid * 320, 320)])

        @pl.when(tid == 0)
        def _():
            pltpu.sync_copy(z1.at[pl.ds(0, 8)], deg_sp.at[pl.ds(_H, 8)])

        plsc.subcore_barrier()
        base_lo = core * _H

        def chunk_a(ci, _):
            off = tid * per_tile + ci * _EC
            pltpu.sync_copy(col_hbm.at[pl.ds(off, _EC)], cbuf)
            for j in range(_EC // 16):
                c = cbuf[pl.ds(j * 16, 16)]
                lc = c - base_lo
                dcol = jnp.where((lc >= 0) & (lc < _H), lc, _H)
                dbuf[pl.ds(j * 16, 16)] = dcol
            pltpu.sync_copy(onesb, deg_sp.at[dbuf], add=True)
            return 0

        lax.fori_loop(0, nchunk, chunk_a, 0)
        plsc.subcore_barrier()

        pltpu.sync_copy(deg_sp.at[pl.ds(tid * 320, 320)], z1)
        pltpu.sync_copy(z1, deg_out.at[pl.ds(core * _H + tid * 320, 320)])

    return k(col_pad)


def _conv_accumulate(y_pad, row_pad, col_pad):
    e_eff = row_pad.shape[0]
    per_tile = e_eff // 16
    nchunk = per_tile // _EC
    mesh = plsc.VectorSubcoreMesh(core_axis_name="c", subcore_axis_name="s")

    @functools.partial(
        pl.kernel,
        out_type=[jax.ShapeDtypeStruct((_NP, 128), jnp.float32)] * 2,
        mesh=mesh,
        scratch_types=[
            pltpu.VMEM((_EC,), jnp.int32),
            pltpu.VMEM((_EC,), jnp.int32),
            pltpu.VMEM((_EC,), jnp.int32),
            pltpu.VMEM((_EC, 128), jnp.float32),
            pltpu.VMEM((128, 128), jnp.float32),
            pltpu.VMEM_SHARED((2 * _H + 16, 128), jnp.float32),
            pltpu.SemaphoreType.DMA,
        ],
    )
    def k(y_hbm, row_hbm, col_hbm, same_out, diff_out,
          rbuf, cbuf, dbuf, gbuf, zbuf, acc, sem):
        core = lax.axis_index("c")
        tid = lax.axis_index("s")

        _zero_block(zbuf, 128)
        for b in range(5):
            pltpu.sync_copy(zbuf, acc.at[pl.ds(tid * 640 + b * 128, 128)])

        @pl.when(tid == 0)
        def _():
            pltpu.sync_copy(zbuf.at[pl.ds(0, 16)], acc.at[pl.ds(2 * _H, 16)])

        plsc.subcore_barrier()

        base_lo = core * _H

        def chunk_body(ci, _):
            off = tid * per_tile + ci * _EC
            pltpu.sync_copy(row_hbm.at[pl.ds(off, _EC)], rbuf)
            pltpu.sync_copy(col_hbm.at[pl.ds(off, _EC)], cbuf)
            for j in range(_EC // 16):
                r = rbuf[pl.ds(j * 16, 16)]
                c = cbuf[pl.ds(j * 16, 16)]
                msame = ((r < _THR) & (c < _THR)) | ((r >= _THR) & (c >= _THR))
                lr = r - base_lo
                valid = (lr >= 0) & (lr < _H)
                dest = jnp.where(valid,
                                 jnp.where(msame, lr, lr + _H),
                                 2 * _H)
                dbuf[pl.ds(j * 16, 16)] = dest
            pltpu.async_copy(y_hbm.at[cbuf], gbuf, sem).wait()
            pltpu.sync_copy(gbuf, acc.at[dbuf], add=True)
            return 0

        lax.fori_loop(0, nchunk, chunk_body, 0)
        plsc.subcore_barrier()

        g0 = core * _H + tid * 320
        pltpu.sync_copy(acc.at[pl.ds(tid * 320, 320)], same_out.at[pl.ds(g0, 320)])
        pltpu.sync_copy(acc.at[pl.ds(_H + tid * 320, 320)], diff_out.at[pl.ds(g0, 320)])

    return k(y_pad, row_pad, col_pad)


def _gather_pairs(xdt, rows_m, cols_m, cnt16):
    e_pad = rows_m.shape[0]
    mesh = plsc.VectorSubcoreMesh(core_axis_name="c", subcore_axis_name="s")

    @functools.partial(
        pl.kernel,
        out_type=[jax.ShapeDtypeStruct((e_pad, 128), jnp.float32)] * 2,
        mesh=mesh,
        scratch_types=[
            pltpu.VMEM((_EC,), jnp.int32),
            pltpu.VMEM((_EC, 128), jnp.float32),
            pltpu.VMEM((16,), jnp.int32),
            pltpu.SemaphoreType.DMA,
        ],
    )
    def k(xdt_hbm, rows_hbm, cols_hbm, cnt_hbm, f1_out, f2_out,
          ibuf, gbuf, cntbuf, sem):
        core = lax.axis_index("c")
        tid = lax.axis_index("s")
        wid = tid * 2 + core

        pltpu.sync_copy(cnt_hbm, cntbuf)
        cnt_s = cntbuf[pl.ds(0, 16)][0]

        nch = (cnt_s + 32 * _EC - 1) // (32 * _EC)
        start = wid * nch * _EC
        trip = jnp.maximum(0, jnp.minimum(nch, (cnt_s - start + _EC - 1) // _EC))

        def chunk_body(ci, _):
            off = start + ci * _EC
            pltpu.sync_copy(rows_hbm.at[pl.ds(off, _EC)], ibuf)
            pltpu.async_copy(xdt_hbm.at[ibuf], gbuf, sem).wait()
            pltpu.sync_copy(gbuf, f1_out.at[pl.ds(off, _EC)])
            pltpu.sync_copy(cols_hbm.at[pl.ds(off, _EC)], ibuf)
            pltpu.async_copy(xdt_hbm.at[ibuf], gbuf, sem).wait()
            pltpu.sync_copy(gbuf, f2_out.at[pl.ds(off, _EC)])
            return 0

        lax.fori_loop(0, trip, chunk_body, 0)

    return k(xdt, rows_m, cols_m, cnt16)


def _transform_body(as_ref, ad_ref, dinv_ref, ws_ref, bs_ref, wd_ref, bd_ref,
                    xs_ref, xd_ref):
    scale = dinv_ref[...]
    a_s = as_ref[...] * scale
    a_d = ad_ref[...] * scale
    dn = (((1,), (1,)), ((), ()))
    xs_ref[...] = lax.dot_general(a_s, ws_ref[...], dn,
                                  preferred_element_type=jnp.float32) + bs_ref[...]
    xd_ref[...] = lax.dot_general(a_d, wd_ref[...], dn,
                                  preferred_element_type=jnp.float32) + bd_ref[...]


def _transform(acc_same, acc_diff, dinv, W_same, b_same, W_diff, b_diff, rb):
    n, d = acc_same.shape
    grid = (n // rb,)
    row_spec = pl.BlockSpec((rb, d), lambda i: (i, 0))
    one_spec = pl.BlockSpec((rb, 1), lambda i: (i, 0))
    w_spec = pl.BlockSpec((d, d), lambda i: (0, 0))
    b_spec = pl.BlockSpec((1, d), lambda i: (0, 0))
    return pl.pallas_call(
        _transform_body,
        grid=grid,
        in_specs=[row_spec, row_spec, one_spec, w_spec, b_spec, w_spec, b_spec],
        out_specs=[row_spec, row_spec],
        out_shape=[jax.ShapeDtypeStruct((n, d), jnp.float32)] * 2,
    )(acc_same, acc_diff, dinv[:, None], W_same, b_same[None, :],
      W_diff, b_diff[None, :])


def _bilinear_body(cnt_ref, f1_ref, f2_ref, w3_ref, bb_ref, eb_ref, *, b, d):
    i = pl.program_id(0)
    nb = (cnt_ref[0] + b - 1) // b

    @pl.when(i < nb)
    def _():
        f1b = f1_ref[...]
        f2b = f2_ref[...]
        acc = jnp.zeros((b, d), jnp.float32) + bb_ref[...]
        for t in range(d):
            a = f1b[:, t:t + 1] * f2b
            acc = acc + jnp.dot(a, w3_ref[t], preferred_element_type=jnp.float32)
        eb_ref[...] = acc


def _bilinear(cnt, f1p, f2p, W3, b_bil, b):
    e_pad, d = f1p.shape
    maxb = e_pad // b

    def edge_idx(i, c):
        nb = lax.div(c[0] + (b - 1), b)
        return (jnp.minimum(i, jnp.maximum(nb - 1, 0)), 0)

    grid_spec = pltpu.PrefetchScalarGridSpec(
        num_scalar_prefetch=1,
        grid=(maxb,),
        in_specs=[
            pl.BlockSpec((b, d), edge_idx),
            pl.BlockSpec((b, d), edge_idx),
            pl.BlockSpec((d, d, d), lambda i, c: (0, 0, 0)),
            pl.BlockSpec((1, d), lambda i, c: (0, 0)),
        ],
        out_specs=pl.BlockSpec((b, d), edge_idx),
    )
    return pl.pallas_call(
        functools.partial(_bilinear_body, b=b, d=d),
        grid_spec=grid_spec,
        out_shape=jax.ShapeDtypeStruct((e_pad, d), jnp.float32),
    )(cnt, f1p, f2p, W3, b_bil[None, :])


def _bil_scatter(eb, rows_m, cols_m, cnt16):
    e_pad = eb.shape[0]
    mesh = plsc.VectorSubcoreMesh(core_axis_name="c", subcore_axis_name="s")

    @functools.partial(
        pl.kernel,
        out_type=[jax.ShapeDtypeStruct((_NP, 128), jnp.float32),
                  jax.ShapeDtypeStruct((_NP,), jnp.float32)],
        mesh=mesh,
        scratch_types=[
            pltpu.VMEM((_EC,), jnp.int32),
            pltpu.VMEM((_EC,), jnp.int32),
            pltpu.VMEM((_EC,), jnp.int32),
            pltpu.VMEM((_EC,), jnp.int32),
            pltpu.VMEM((_EC,), jnp.float32),
            pltpu.VMEM((_EC, 128), jnp.float32),
            pltpu.VMEM((128, 128), jnp.float32),
            pltpu.VMEM((320,), jnp.float32),
            pltpu.VMEM((320,), jnp.float32),
            pltpu.VMEM((16,), jnp.int32),
            pltpu.VMEM_SHARED((_H + 8, 128), jnp.float32),
            pltpu.VMEM_SHARED((_H + 8,), jnp.float32),
            pltpu.SemaphoreType.DMA,
        ],
    )
    def k(eb_hbm, rows_hbm, cols_hbm, cnt_hbm, nb_out, cnts_out,
          rbuf, cbuf, dbr, dbc, onesb, gbuf, zbuf, z1, cbounce, cntbuf,
          nb_acc, cnt_acc, sem):
        core = lax.axis_index("c")
        tid = lax.axis_index("s")
        ones = jnp.ones((16,), jnp.float32)

        _zero_block(zbuf, 128)
        for j in range(_EC // 16):
            onesb[pl.ds(j * 16, 16)] = ones
        for j in range(320 // 16):
            z1[pl.ds(j * 16, 16)] = jnp.zeros((16,), jnp.float32)

        pltpu.sync_copy(zbuf, nb_acc.at[pl.ds(tid * 320, 128)])
        pltpu.sync_copy(zbuf, nb_acc.at[pl.ds(tid * 320 + 128, 128)])
        pltpu.sync_copy(zbuf.at[pl.ds(0, 64)], nb_acc.at[pl.ds(tid * 320 + 256, 64)])
        pltpu.sync_copy(z1, cnt_acc.at[pl.ds(tid * 320, 320)])

        @pl.when(tid == 0)
        def _():
            pltpu.sync_copy(zbuf.at[pl.ds(0, 8)], nb_acc.at[pl.ds(_H, 8)])
            pltpu.sync_copy(z1.at[pl.ds(0, 8)], cnt_acc.at[pl.ds(_H, 8)])

        pltpu.sync_copy(cnt_hbm, cntbuf)
        cv = cntbuf[pl.ds(0, 16)]
        cnt_s = cv[0]
        plsc.subcore_barrier()

        nch = (cnt_s + 16 * _EC - 1) // (16 * _EC)
        start = tid * nch * _EC
        trip = jnp.maximum(0, jnp.minimum(nch, (cnt_s - start + _EC - 1) // _EC))
        base_lo = core * _H

        def chunk_body(ci, _):
            off = start + ci * _EC
            pltpu.sync_copy(rows_hbm.at[pl.ds(off, _EC)], rbuf)
            pltpu.sync_copy(cols_hbm.at[pl.ds(off, _EC)], cbuf)
            pltpu.sync_copy(eb_hbm.at[pl.ds(off, _EC)], gbuf)
            cvl = cntbuf[pl.ds(0, 16)]
            for j in range(_EC // 16):
                gpos = off + j * 16 + lax.broadcasted_iota(jnp.int32, (16,), 0)
                ev = gpos < cvl
                r = rbuf[pl.ds(j * 16, 16)]
                c = cbuf[pl.ds(j * 16, 16)]
                lr = r - base_lo
                dr = jnp.where(ev & (lr >= 0) & (lr < _H), lr, _H)
                lc = c - base_lo
                dc = jnp.where(ev & (lc >= 0) & (lc < _H), lc, _H)
                dbr[pl.ds(j * 16, 16)] = dr
                dbc[pl.ds(j * 16, 16)] = dc
            pltpu.sync_copy(gbuf, nb_acc.at[dbr], add=True)
            pltpu.sync_copy(gbuf, nb_acc.at[dbc], add=True)
            pltpu.sync_copy(onesb, cnt_acc.at[dbr], add=True)
            pltpu.sync_copy(onesb, cnt_acc.at[dbc], add=True)
            return 0

        lax.fori_loop(0, trip, chunk_body, 0)
        plsc.subcore_barrier()

        g0 = core * _H + tid * 320
        pltpu.sync_copy(nb_acc.at[pl.ds(tid * 320, 320)], nb_out.at[pl.ds(g0, 320)])
        pltpu.sync_copy(cnt_acc.at[pl.ds(tid * 320, 320)], cbounce)
        pltpu.sync_copy(cbounce, cnts_out.at[pl.ds(g0, 320)])

    return k(eb, rows_m, cols_m, cnt16)


def _fuse_body(xs_ref, xd_ref, nb_ref, cnt_ref, gw_ref, out_ref):
    gate = 1.0 / (1.0 + jnp.exp(-gw_ref[...]))
    counts = jnp.maximum(cnt_ref[...], 1.0)
    bf = nb_ref[...] / counts
    leaky = jnp.where(bf >= 0, bf, 0.01 * bf)
    out_ref[...] = xs_ref[...] + gate * leaky + (1.0 - gate) * xd_ref[...]


def _fuse(x_same_t, x_diff_t, node_bil, counts, gate_weight, rb):
    n, d = x_same_t.shape
    row_spec = pl.BlockSpec((rb, d), lambda i: (i, 0))
    one_spec = pl.BlockSpec((rb, 1), lambda i: (i, 0))
    g_spec = pl.BlockSpec((1, d), lambda i: (0, 0))
    return pl.pallas_call(
        _fuse_body,
        grid=(n // rb,),
        in_specs=[row_spec, row_spec, row_spec, one_spec, g_spec],
        out_specs=row_spec,
        out_shape=jax.ShapeDtypeStruct((n, d), jnp.float32),
    )(x_same_t, x_diff_t, node_bil, counts[:, None], gate_weight[None, :])


def kernel(x, edge_index, W_same, b_same, W_diff, b_diff, W_bil, b_bil,
           gate_weight):
    n, d = x.shape
    e = edge_index.shape[1]
    rb = 1024
    row = edge_index[0]
    col = edge_index[1]

    e_eff = 161792
    row_pad = jnp.pad(row, (0, e_eff - e), constant_values=_N)
    col_pad = jnp.pad(col, (0, e_eff - e), constant_values=_N)
    deg = _deg_kernel(col_pad)
    dinv = deg ** -0.5

    x_pad = jnp.pad(x, ((0, _NP - n), (0, 0)))
    y_pad = dinv[:, None] * x_pad
    acc_same, acc_diff = _conv_accumulate(y_pad, row_pad, col_pad)

    x_same_t, x_diff_t = _transform(acc_same, acc_diff, dinv,
                                    W_same, b_same, W_diff, b_diff, rb=rb)

    m_md = (row < _THR) & (col >= _THR)
    order = jnp.argsort(jnp.logical_not(m_md), stable=True)
    e_pad = ((e + _BIL_B - 1) // _BIL_B) * _BIL_B
    pad = e_pad - e
    rows_m = jnp.pad(row[order], (0, pad))
    cols_m = jnp.pad(col[order], (0, pad))
    cnt = jnp.sum(m_md).astype(jnp.int32)[None]
    cnt16 = jnp.broadcast_to(cnt[0], (16,))
    f1, f2 = _gather_pairs(x_diff_t, rows_m, cols_m, cnt16)

    W3 = jnp.transpose(W_bil, (1, 2, 0))
    eb = _bilinear(cnt, f1, f2, W3, b_bil, _BIL_B)

    node_bil, counts = _bil_scatter(eb, rows_m, cols_m, cnt16)

    out = _fuse(x_same_t, x_diff_t, node_bil, counts, gate_weight, rb=rb)
    return out[:n]


# speedup vs baseline: 2.9745x; 1.0596x over previous
import functools

import jax
import jax.numpy as jnp
from jax import lax
from jax.experimental import pallas as pl
from jax.experimental.pallas import tpu as pltpu
from jax.experimental.pallas import tpu_sc as plsc

_THR = 812
_BIL_B = 1024
_N = 10000
_NP = 10240
_H = 5120
_EC = 128


def _zero_block(zbuf, rows):
    zeros = jnp.zeros((16,), jnp.float32)

    def body(t, _):
        i = t // 8
        j = (t % 8) * 16
        zbuf[i, pl.ds(j, 16)] = zeros
        return 0

    lax.fori_loop(0, rows * 8, body, 0)


def _deg_kernel(col_pad):
    e_eff = col_pad.shape[0]
    per_tile = e_eff // 16
    nchunk = per_tile // _EC
    mesh = plsc.VectorSubcoreMesh(core_axis_name="c", subcore_axis_name="s")

    @functools.partial(
        pl.kernel,
        out_type=jax.ShapeDtypeStruct((_NP,), jnp.float32),
        mesh=mesh,
        scratch_types=[
            pltpu.VMEM((_EC,), jnp.int32),
            pltpu.VMEM((_EC,), jnp.int32),
            pltpu.VMEM((_EC,), jnp.float32),
            pltpu.VMEM((320,), jnp.float32),
            pltpu.VMEM_SHARED((_H + 8,), jnp.float32),
        ],
    )
    def k(col_hbm, deg_out, cbuf, dbuf, onesb, z1, deg_sp):
        core = lax.axis_index("c")
        tid = lax.axis_index("s")
        zeros = jnp.zeros((16,), jnp.float32)
        ones = jnp.ones((16,), jnp.float32)

        for j in range(320 // 16):
            z1[pl.ds(j * 16, 16)] = zeros
        for j in range(_EC // 16):
            onesb[pl.ds(j * 16, 16)] = ones
        pltpu.sync_copy(z1, deg_sp.at[pl.ds(tid * 320, 320)])

        @pl.when(tid == 0)
        def _():
            pltpu.sync_copy(z1.at[pl.ds(0, 8)], deg_sp.at[pl.ds(_H, 8)])

        plsc.subcore_barrier()
        base_lo = core * _H

        def chunk_a(ci, _):
            off = tid * per_tile + ci * _EC
            pltpu.sync_copy(col_hbm.at[pl.ds(off, _EC)], cbuf)
            for j in range(_EC // 16):
                c = cbuf[pl.ds(j * 16, 16)]
                lc = c - base_lo
                dcol = jnp.where((lc >= 0) & (lc < _H), lc, _H)
                dbuf[pl.ds(j * 16, 16)] = dcol
            pltpu.sync_copy(onesb, deg_sp.at[dbuf], add=True)
            return 0

        lax.fori_loop(0, nchunk, chunk_a, 0)
        plsc.subcore_barrier()

        pltpu.sync_copy(deg_sp.at[pl.ds(tid * 320, 320)], z1)
        pltpu.sync_copy(z1, deg_out.at[pl.ds(core * _H + tid * 320, 320)])

    return k(col_pad)


def _conv_accumulate(y_pad, row_pad, col_pad):
    e_eff = row_pad.shape[0]
    per_tile = e_eff // 16
    nchunk = per_tile // _EC
    mesh = plsc.VectorSubcoreMesh(core_axis_name="c", subcore_axis_name="s")

    @functools.partial(
        pl.kernel,
        out_type=[jax.ShapeDtypeStruct((_NP, 128), jnp.float32)] * 2,
        mesh=mesh,
        scratch_types=[
            pltpu.VMEM((_EC,), jnp.int32),
            pltpu.VMEM((_EC,), jnp.int32),
            pltpu.VMEM((_EC,), jnp.int32),
            pltpu.VMEM((_EC, 128), jnp.float32),
            pltpu.VMEM((_EC,), jnp.int32),
            pltpu.VMEM((_EC,), jnp.int32),
            pltpu.VMEM((_EC,), jnp.int32),
            pltpu.VMEM((_EC, 128), jnp.float32),
            pltpu.VMEM((64, 128), jnp.float32),
            pltpu.VMEM_SHARED((2 * _H + 2, 128), jnp.float32),
            pltpu.SemaphoreType.DMA,
            pltpu.SemaphoreType.DMA,
        ],
    )
    def k(y_hbm, row_hbm, col_hbm, same_out, diff_out,
          rbuf, cbuf, dbuf, gbuf, rbuf2, cbuf2, dbuf2, gbuf2, zbuf, acc,
          sem, sem2):
        core = lax.axis_index("c")
        tid = lax.axis_index("s")

        _zero_block(zbuf, 64)
        for b in range(10):
            pltpu.sync_copy(zbuf, acc.at[pl.ds(tid * 640 + b * 64, 64)])

        @pl.when(tid == 0)
        def _():
            pltpu.sync_copy(zbuf.at[pl.ds(0, 2)], acc.at[pl.ds(2 * _H, 2)])

        plsc.subcore_barrier()

        base_lo = core * _H

        def route(rb, cb, db):
            for j in range(_EC // 16):
                r = rb[pl.ds(j * 16, 16)]
                c = cb[pl.ds(j * 16, 16)]
                msame = ((r < _THR) & (c < _THR)) | ((r >= _THR) & (c >= _THR))
                lr = r - base_lo
                valid = (lr >= 0) & (lr < _H)
                dest = jnp.where(valid,
                                 jnp.where(msame, lr, lr + _H),
                                 2 * _H)
                db[pl.ds(j * 16, 16)] = dest

        def pair_body(ci, _):
            off = tid * per_tile + ci * 2 * _EC
            pltpu.sync_copy(row_hbm.at[pl.ds(off, _EC)], rbuf)
            pltpu.sync_copy(col_hbm.at[pl.ds(off, _EC)], cbuf)
            d0 = pltpu.async_copy(y_hbm.at[cbuf], gbuf, sem)
            pltpu.sync_copy(row_hbm.at[pl.ds(off + _EC, _EC)], rbuf2)
            pltpu.sync_copy(col_hbm.at[pl.ds(off + _EC, _EC)], cbuf2)
            d1 = pltpu.async_copy(y_hbm.at[cbuf2], gbuf2, sem2)
            route(rbuf, cbuf, dbuf)
            route(rbuf2, cbuf2, dbuf2)
            d0.wait()
            pltpu.sync_copy(gbuf, acc.at[dbuf], add=True)
            d1.wait()
            pltpu.sync_copy(gbuf2, acc.at[dbuf2], add=True)
            return 0

        lax.fori_loop(0, nchunk // 2, pair_body, 0)

        toff = tid * per_tile + (nchunk - 1) * _EC
        pltpu.sync_copy(row_hbm.at[pl.ds(toff, _EC)], rbuf)
        pltpu.sync_copy(col_hbm.at[pl.ds(toff, _EC)], cbuf)
        route(rbuf, cbuf, dbuf)
        pltpu.async_copy(y_hbm.at[cbuf], gbuf, sem).wait()
        pltpu.sync_copy(gbuf, acc.at[dbuf], add=True)
        plsc.subcore_barrier()

        g0 = core * _H + tid * 320
        pltpu.sync_copy(acc.at[pl.ds(tid * 320, 320)], same_out.at[pl.ds(g0, 320)])
        pltpu.sync_copy(acc.at[pl.ds(_H + tid * 320, 320)], diff_out.at[pl.ds(g0, 320)])

    return k(y_pad, row_pad, col_pad)


def _gather_pairs(xdt, rows_m, cols_m, cnt16):
    e_pad = rows_m.shape[0]
    mesh = plsc.VectorSubcoreMesh(core_axis_name="c", subcore_axis_name="s")

    @functools.partial(
        pl.kernel,
        out_type=[jax.ShapeDtypeStruct((e_pad, 128), jnp.float32)] * 2,
        mesh=mesh,
        scratch_types=[
            pltpu.VMEM((_EC,), jnp.int32),
            pltpu.VMEM((_EC, 128), jnp.float32),
            pltpu.VMEM((16,), jnp.int32),
            pltpu.SemaphoreType.DMA,
        ],
    )
    def k(xdt_hbm, rows_hbm, cols_hbm, cnt_hbm, f1_out, f2_out,
          ibuf, gbuf, cntbuf, sem):
        core = lax.axis_index("c")
        tid = lax.axis_index("s")
        wid = tid * 2 + core

        pltpu.sync_copy(cnt_hbm, cntbuf)
        cnt_s = cntbuf[pl.ds(0, 16)][0]

        nch = (cnt_s + 32 * _EC - 1) // (32 * _EC)
        start = wid * nch * _EC
        trip = jnp.maximum(0, jnp.minimum(nch, (cnt_s - start + _EC - 1) // _EC))

        def chunk_body(ci, _):
            off = start + ci * _EC
            pltpu.sync_copy(rows_hbm.at[pl.ds(off, _EC)], ibuf)
            pltpu.async_copy(xdt_hbm.at[ibuf], gbuf, sem).wait()
            pltpu.sync_copy(gbuf, f1_out.at[pl.ds(off, _EC)])
            pltpu.sync_copy(cols_hbm.at[pl.ds(off, _EC)], ibuf)
            pltpu.async_copy(xdt_hbm.at[ibuf], gbuf, sem).wait()
            pltpu.sync_copy(gbuf, f2_out.at[pl.ds(off, _EC)])
            return 0

        lax.fori_loop(0, trip, chunk_body, 0)

    return k(xdt, rows_m, cols_m, cnt16)


def _transform_body(as_ref, ad_ref, dinv_ref, ws_ref, bs_ref, wd_ref, bd_ref,
                    xs_ref, xd_ref):
    scale = dinv_ref[...]
    a_s = as_ref[...] * scale
    a_d = ad_ref[...] * scale
    dn = (((1,), (1,)), ((), ()))
    xs_ref[...] = lax.dot_general(a_s, ws_ref[...], dn,
                                  preferred_element_type=jnp.float32) + bs_ref[...]
    xd_ref[...] = lax.dot_general(a_d, wd_ref[...], dn,
                                  preferred_element_type=jnp.float32) + bd_ref[...]


def _transform(acc_same, acc_diff, dinv, W_same, b_same, W_diff, b_diff, rb):
    n, d = acc_same.shape
    grid = (n // rb,)
    row_spec = pl.BlockSpec((rb, d), lambda i: (i, 0))
    one_spec = pl.BlockSpec((rb, 1), lambda i: (i, 0))
    w_spec = pl.BlockSpec((d, d), lambda i: (0, 0))
    b_spec = pl.BlockSpec((1, d), lambda i: (0, 0))
    return pl.pallas_call(
        _transform_body,
        grid=grid,
        in_specs=[row_spec, row_spec, one_spec, w_spec, b_spec, w_spec, b_spec],
        out_specs=[row_spec, row_spec],
        out_shape=[jax.ShapeDtypeStruct((n, d), jnp.float32)] * 2,
    )(acc_same, acc_diff, dinv[:, None], W_same, b_same[None, :],
      W_diff, b_diff[None, :])


def _bilinear_body(cnt_ref, f1_ref, f2_ref, w3_ref, bb_ref, eb_ref, *, b, d):
    i = pl.program_id(0)
    nb = (cnt_ref[0] + b - 1) // b

    @pl.when(i < nb)
    def _():
        f1b = f1_ref[...]
        f2b = f2_ref[...]
        acc = jnp.zeros((b, d), jnp.float32) + bb_ref[...]
        for t in range(d):
            a = f1b[:, t:t + 1] * f2b
            acc = acc + jnp.dot(a, w3_ref[t], preferred_element_type=jnp.float32)
        eb_ref[...] = acc


def _bilinear(cnt, f1p, f2p, W3, b_bil, b):
    e_pad, d = f1p.shape
    maxb = e_pad // b

    def edge_idx(i, c):
        nb = lax.div(c[0] + (b - 1), b)
        return (jnp.minimum(i, jnp.maximum(nb - 1, 0)), 0)

    grid_spec = pltpu.PrefetchScalarGridSpec(
        num_scalar_prefetch=1,
        grid=(maxb,),
        in_specs=[
            pl.BlockSpec((b, d), edge_idx),
            pl.BlockSpec((b, d), edge_idx),
            pl.BlockSpec((d, d, d), lambda i, c: (0, 0, 0)),
            pl.BlockSpec((1, d), lambda i, c: (0, 0)),
        ],
        out_specs=pl.BlockSpec((b, d), edge_idx),
    )
    return pl.pallas_call(
        functools.partial(_bilinear_body, b=b, d=d),
        grid_spec=grid_spec,
        out_shape=jax.ShapeDtypeStruct((e_pad, d), jnp.float32),
    )(cnt, f1p, f2p, W3, b_bil[None, :])


def _bil_scatter(eb, rows_m, cols_m, cnt16):
    e_pad = eb.shape[0]
    mesh = plsc.VectorSubcoreMesh(core_axis_name="c", subcore_axis_name="s")

    @functools.partial(
        pl.kernel,
        out_type=[jax.ShapeDtypeStruct((_NP, 128), jnp.float32),
                  jax.ShapeDtypeStruct((_NP,), jnp.float32)],
        mesh=mesh,
        scratch_types=[
            pltpu.VMEM((_EC,), jnp.int32),
            pltpu.VMEM((_EC,), jnp.int32),
            pltpu.VMEM((_EC,), jnp.int32),
            pltpu.VMEM((_EC,), jnp.int32),
            pltpu.VMEM((_EC,), jnp.float32),
            pltpu.VMEM((_EC, 128), jnp.float32),
            pltpu.VMEM((128, 128), jnp.float32),
            pltpu.VMEM((320,), jnp.float32),
            pltpu.VMEM((320,), jnp.float32),
            pltpu.VMEM((16,), jnp.int32),
            pltpu.VMEM_SHARED((_H + 8, 128), jnp.float32),
            pltpu.VMEM_SHARED((_H + 8,), jnp.float32),
            pltpu.SemaphoreType.DMA,
        ],
    )
    def k(eb_hbm, rows_hbm, cols_hbm, cnt_hbm, nb_out, cnts_out,
          rbuf, cbuf, dbr, dbc, onesb, gbuf, zbuf, z1, cbounce, cntbuf,
          nb_acc, cnt_acc, sem):
        core = lax.axis_index("c")
        tid = lax.axis_index("s")
        ones = jnp.ones((16,), jnp.float32)

        _zero_block(zbuf, 128)
        for j in range(_EC // 16):
            onesb[pl.ds(j * 16, 16)] = ones
        for j in range(320 // 16):
            z1[pl.ds(j * 16, 16)] = jnp.zeros((16,), jnp.float32)

        pltpu.sync_copy(zbuf, nb_acc.at[pl.ds(tid * 320, 128)])
        pltpu.sync_copy(zbuf, nb_acc.at[pl.ds(tid * 320 + 128, 128)])
        pltpu.sync_copy(zbuf.at[pl.ds(0, 64)], nb_acc.at[pl.ds(tid * 320 + 256, 64)])
        pltpu.sync_copy(z1, cnt_acc.at[pl.ds(tid * 320, 320)])

        @pl.when(tid == 0)
        def _():
            pltpu.sync_copy(zbuf.at[pl.ds(0, 8)], nb_acc.at[pl.ds(_H, 8)])
            pltpu.sync_copy(z1.at[pl.ds(0, 8)], cnt_acc.at[pl.ds(_H, 8)])

        pltpu.sync_copy(cnt_hbm, cntbuf)
        cv = cntbuf[pl.ds(0, 16)]
        cnt_s = cv[0]
        plsc.subcore_barrier()

        nch = (cnt_s + 16 * _EC - 1) // (16 * _EC)
        start = tid * nch * _EC
        trip = jnp.maximum(0, jnp.minimum(nch, (cnt_s - start + _EC - 1) // _EC))
        base_lo = core * _H

        def chunk_body(ci, _):
            off = start + ci * _EC
            pltpu.sync_copy(rows_hbm.at[pl.ds(off, _EC)], rbuf)
            pltpu.sync_copy(cols_hbm.at[pl.ds(off, _EC)], cbuf)
            pltpu.sync_copy(eb_hbm.at[pl.ds(off, _EC)], gbuf)
            cvl = cntbuf[pl.ds(0, 16)]
            for j in range(_EC // 16):
                gpos = off + j * 16 + lax.broadcasted_iota(jnp.int32, (16,), 0)
                ev = gpos < cvl
                r = rbuf[pl.ds(j * 16, 16)]
                c = cbuf[pl.ds(j * 16, 16)]
                lr = r - base_lo
                dr = jnp.where(ev & (lr >= 0) & (lr < _H), lr, _H)
                lc = c - base_lo
                dc = jnp.where(ev & (lc >= 0) & (lc < _H), lc, _H)
                dbr[pl.ds(j * 16, 16)] = dr
                dbc[pl.ds(j * 16, 16)] = dc
            pltpu.sync_copy(gbuf, nb_acc.at[dbr], add=True)
            pltpu.sync_copy(gbuf, nb_acc.at[dbc], add=True)
            pltpu.sync_copy(onesb, cnt_acc.at[dbr], add=True)
            pltpu.sync_copy(onesb, cnt_acc.at[dbc], add=True)
            return 0

        lax.fori_loop(0, trip, chunk_body, 0)
        plsc.subcore_barrier()

        g0 = core * _H + tid * 320
        pltpu.sync_copy(nb_acc.at[pl.ds(tid * 320, 320)], nb_out.at[pl.ds(g0, 320)])
        pltpu.sync_copy(cnt_acc.at[pl.ds(tid * 320, 320)], cbounce)
        pltpu.sync_copy(cbounce, cnts_out.at[pl.ds(g0, 320)])

    return k(eb, rows_m, cols_m, cnt16)


def _fuse_body(xs_ref, xd_ref, nb_ref, cnt_ref, gw_ref, out_ref):
    gate = 1.0 / (1.0 + jnp.exp(-gw_ref[...]))
    counts = jnp.maximum(cnt_ref[...], 1.0)
    bf = nb_ref[...] / counts
    leaky = jnp.where(bf >= 0, bf, 0.01 * bf)
    out_ref[...] = xs_ref[...] + gate * leaky + (1.0 - gate) * xd_ref[...]


def _fuse(x_same_t, x_diff_t, node_bil, counts, gate_weight, rb):
    n, d = x_same_t.shape
    row_spec = pl.BlockSpec((rb, d), lambda i: (i, 0))
    one_spec = pl.BlockSpec((rb, 1), lambda i: (i, 0))
    g_spec = pl.BlockSpec((1, d), lambda i: (0, 0))
    return pl.pallas_call(
        _fuse_body,
        grid=(n // rb,),
        in_specs=[row_spec, row_spec, row_spec, one_spec, g_spec],
        out_specs=row_spec,
        out_shape=jax.ShapeDtypeStruct((n, d), jnp.float32),
    )(x_same_t, x_diff_t, node_bil, counts[:, None], gate_weight[None, :])


def kernel(x, edge_index, W_same, b_same, W_diff, b_diff, W_bil, b_bil,
           gate_weight):
    n, d = x.shape
    e = edge_index.shape[1]
    rb = 1024
    row = edge_index[0]
    col = edge_index[1]

    e_eff = 161792
    row_pad = jnp.pad(row, (0, e_eff - e), constant_values=_N)
    col_pad = jnp.pad(col, (0, e_eff - e), constant_values=_N)
    deg = _deg_kernel(col_pad)
    dinv = deg ** -0.5

    x_pad = jnp.pad(x, ((0, _NP - n), (0, 0)))
    y_pad = dinv[:, None] * x_pad
    acc_same, acc_diff = _conv_accumulate(y_pad, row_pad, col_pad)

    x_same_t, x_diff_t = _transform(acc_same, acc_diff, dinv,
                                    W_same, b_same, W_diff, b_diff, rb=rb)

    m_md = (row < _THR) & (col >= _THR)
    order = jnp.argsort(jnp.logical_not(m_md), stable=True)
    e_pad = ((e + _BIL_B - 1) // _BIL_B) * _BIL_B
    pad = e_pad - e
    rows_m = jnp.pad(row[order], (0, pad))
    cols_m = jnp.pad(col[order], (0, pad))
    cnt = jnp.sum(m_md).astype(jnp.int32)[None]
    cnt16 = jnp.broadcast_to(cnt[0], (16,))
    f1, f2 = _gather_pairs(x_diff_t, rows_m, cols_m, cnt16)

    W3 = jnp.transpose(W_bil, (1, 2, 0))
    eb = _bilinear(cnt, f1, f2, W3, b_bil, _BIL_B)

    node_bil, counts = _bil_scatter(eb, rows_m, cols_m, cnt16)

    out = _fuse(x_same_t, x_diff_t, node_bil, counts, gate_weight, rb=rb)
    return out[:n]
